# baseline (speedup 1.0000x reference)
import jax
import jax.numpy as jnp
from jax import lax
from jax.experimental import pallas as pl
from jax.experimental.pallas import tpu as pltpu
from jax.experimental.pallas import tpu_sc as plsc

N = 10000
E = 320000
HDIM = 128
NP = 10000
WEXT = 144
NEG = 0.2
BN_EPS = 1e-5
G = 16
C = 10

NCORES = 2
NSUB = 16
NTILES = NCORES * NSUB
EPT = E // NTILES
K = 80
NCHUNK = EPT // K
CPG = 25
NGRP = NCHUNK // CPG
RPS = NP // NSUB
BLK = 2000
NBLK = NP // BLK


def _sc_edge_body(hext_hbm, alphas_hbm, alphad_hbm, alm_hbm, src_hbm,
                  dst_hbm, acc_hbm, alphas_v, alphad_v, alm_v, src_v, dst_v,
                  rows_v, ex_v, acc_sh, gsem):
    c = lax.axis_index("c")
    s = lax.axis_index("s")
    wid = s * NCORES + c

    pltpu.sync_copy(alphas_hbm, alphas_v)
    pltpu.sync_copy(alphad_hbm, alphad_v)
    pltpu.sync_copy(alm_hbm, alm_v)

    def zrow(i, carry):
        for cv in range(WEXT // 16):
            rows_v[i, pl.ds(cv * 16, 16)] = jnp.zeros((16,), jnp.float32)
        return carry

    lax.fori_loop(0, K, zrow, 0)
    nzc = RPS // K
    rem = RPS - nzc * K
    for j in range(nzc):
        pltpu.sync_copy(rows_v, acc_sh.at[pl.ds(s * RPS + j * K, K)])
    pltpu.sync_copy(rows_v.at[pl.ds(0, rem)],
                    acc_sh.at[pl.ds(s * RPS + nzc * K, rem)])
    plsc.subcore_barrier()

    gb = alm_v[pl.ds(0, 16)] + alm_v[pl.ds(512, 16)]
    gbound = jnp.where(gb > 0, gb, NEG * gb)

    def group(gix, carry):
        pltpu.sync_copy(src_hbm.at[wid, pl.ds(gix * CPG, CPG)], src_v)
        pltpu.sync_copy(dst_hbm.at[wid, pl.ds(gix * CPG, CPG)], dst_v)

        def chunk(cix, carry1):
            pltpu.async_copy(hext_hbm.at[src_v.at[cix]], rows_v, gsem).wait()
            for v5 in range(K // 16):
                sidx = src_v[cix, pl.ds(v5 * 16, 16)]
                didx = dst_v[cix, pl.ds(v5 * 16, 16)]
                av = plsc.load_gather(alphas_v, [sidx])
                bv = plsc.load_gather(alphad_v, [didx])
                z = av + bv
                z = jnp.where(z > 0, z, NEG * z)
                ex_v[pl.ds(v5 * 16, 16)] = jnp.exp(z - gbound)
            def srow(r4, carry2):
                for u in range(8):
                    r = r4 * 8 + u
                    bex = plsc.load_gather(ex_v,
                                           [jnp.full((16,), r, jnp.int32)])
                    for cv in range(WEXT // 16):
                        rows_v[r, pl.ds(cv * 16, 16)] = (
                            rows_v[r, pl.ds(cv * 16, 16)] * bex)
                return carry2
            lax.fori_loop(0, K // 8, srow, 0)
            pltpu.sync_copy(rows_v, acc_sh.at[dst_v.at[cix]], add=True)
            return carry1

        lax.fori_loop(0, CPG, chunk, 0)
        return carry

    lax.fori_loop(0, NGRP, group, 0)
    plsc.subcore_barrier()

    pltpu.sync_copy(acc_sh.at[pl.ds(s * RPS, RPS)],
                    acc_hbm.at[c, pl.ds(s * RPS, RPS)])


def _sc_edge(hext, alphas, alphad, alm, src3, dst3):
    mesh = plsc.VectorSubcoreMesh(core_axis_name="c", subcore_axis_name="s",
                                  num_cores=NCORES, num_subcores=NSUB)
    f = pl.kernel(
        _sc_edge_body,
        out_type=jax.ShapeDtypeStruct((NCORES, NP, WEXT), jnp.float32),
        mesh=mesh,
        compiler_params=pltpu.CompilerParams(needs_layout_passes=False,
                                             use_tc_tiling_on_sc=False),
        scratch_types=[
            pltpu.VMEM((NP,), jnp.float32),
            pltpu.VMEM((NP,), jnp.float32),
            pltpu.VMEM((G * 64,), jnp.float32),
            pltpu.VMEM((CPG, K), jnp.int32),
            pltpu.VMEM((CPG, K), jnp.int32),
            pltpu.VMEM((K, WEXT), jnp.float32),
            pltpu.VMEM((128,), jnp.float32),
            pltpu.VMEM_SHARED((NP, WEXT), jnp.float32),
            pltpu.SemaphoreType.DMA,
        ],
    )
    return f(hext, alphas, alphad, alm, src3, dst3)


def _epilogue(a0, a1, p):
    u = a0[:, :HDIM] + a1[:, :HDIM]
    dn = a0[:, HDIM:HDIM + 1] + a1[:, HDIM:HDIM + 1]
    u = u / (dn + 1e-30)
    u = u + p[0:1, :]
    u = (u - p[3:4, :]) * lax.rsqrt(p[4:5, :] + BN_EPS) * p[1:2, :] + p[2:3, :]
    return jnp.maximum(u, 0.0)


def _hext_alpha(i, t, w, avec, hext_ref, al_ref, alm_ref):
    h = jnp.dot(t, w, preferred_element_type=jnp.float32)
    hext_ref[...] = jnp.concatenate(
        [h, jnp.ones((h.shape[0], 1), jnp.float32),
         jnp.zeros((h.shape[0], WEXT - HDIM - 1), jnp.float32)], axis=-1)
    al = lax.dot_general(avec, h, (((1,), (1,)), ((), ())),
                         preferred_element_type=jnp.float32)
    al_ref[...] = al[None]

    @pl.when(i == 0)
    def _():
        alm_ref[...] = jnp.full((8, HDIM), -3e38, jnp.float32)

    bmax = jnp.max(al, axis=1, keepdims=True)
    bmax8 = jnp.concatenate(
        [jnp.broadcast_to(bmax[0:1], (4, HDIM)),
         jnp.broadcast_to(bmax[1:2], (4, HDIM))], axis=0)
    alm_ref[...] = jnp.maximum(alm_ref[...], bmax8)


def _tc_first_body(x_ref, w_ref, avec_ref, hext_ref, al_ref, alm_ref):
    _hext_alpha(pl.program_id(0), x_ref[...], w_ref[...], avec_ref[...],
                hext_ref, al_ref, alm_ref)


def _tc_layer_body(acc_ref, p_ref, w_ref, avec_ref, hext_ref, al_ref,
                   alm_ref):
    t = _epilogue(acc_ref[0], acc_ref[1], p_ref[...])
    _hext_alpha(pl.program_id(0), t, w_ref[...], avec_ref[...],
                hext_ref, al_ref, alm_ref)


def _tc_pool_body(acc_ref, p_ref, batch_ref, wc_ref, bc_ref, out_ref,
                  sums_ref):
    i = pl.program_id(0)

    @pl.when(i == 0)
    def _():
        sums_ref[...] = jnp.zeros_like(sums_ref)

    t = _epilogue(acc_ref[0], acc_ref[1], p_ref[...])
    text = jnp.concatenate(
        [t, jnp.ones((t.shape[0], 1), jnp.float32),
         jnp.zeros((t.shape[0], WEXT - HDIM - 1), jnp.float32)], axis=-1)
    oh = (batch_ref[0] == jnp.arange(G, dtype=jnp.int32)[:, None])
    sums_ref[...] += lax.dot_general(
        oh.astype(jnp.float32), text, (((1,), (0,)), ((), ())),
        preferred_element_type=jnp.float32)

    @pl.when(i == pl.num_programs(0) - 1)
    def _():
        sums = sums_ref[...]
        pooled = sums[:, :HDIM] / jnp.clip(sums[:, HDIM:HDIM + 1], 1.0, None)
        out_ref[...] = (jnp.dot(pooled, wc_ref[...],
                                preferred_element_type=jnp.float32)
                        + bc_ref[...])


_DENSE_OUT = [
    jax.ShapeDtypeStruct((NP, WEXT), jnp.float32),
    jax.ShapeDtypeStruct((NBLK, 2, BLK), jnp.float32),
    jax.ShapeDtypeStruct((8, HDIM), jnp.float32),
]
_DENSE_OUT_SPECS = [
    pl.BlockSpec((BLK, WEXT), lambda i: (i, 0)),
    pl.BlockSpec((1, 2, BLK), lambda i: (i, 0, 0)),
    pl.BlockSpec((8, HDIM), lambda i: (0, 0)),
]


def _tc_first(x, w, avec):
    return pl.pallas_call(
        _tc_first_body,
        grid=(NBLK,),
        in_specs=[
            pl.BlockSpec((BLK, HDIM), lambda i: (i, 0)),
            pl.BlockSpec((HDIM, HDIM), lambda i: (0, 0)),
            pl.BlockSpec((2, HDIM), lambda i: (0, 0)),
        ],
        out_specs=_DENSE_OUT_SPECS,
        out_shape=_DENSE_OUT,
    )(x, w, avec)


def _tc_layer(acc, p, w, avec):
    return pl.pallas_call(
        _tc_layer_body,
        grid=(NBLK,),
        in_specs=[
            pl.BlockSpec((2, BLK, WEXT), lambda i: (0, i, 0)),
            pl.BlockSpec((5, HDIM), lambda i: (0, 0)),
            pl.BlockSpec((HDIM, HDIM), lambda i: (0, 0)),
            pl.BlockSpec((2, HDIM), lambda i: (0, 0)),
        ],
        out_specs=_DENSE_OUT_SPECS,
        out_shape=_DENSE_OUT,
    )(acc, p, w, avec)


def _tc_pool(acc, p, batch3, wc_pad, bc_pad):
    return pl.pallas_call(
        _tc_pool_body,
        grid=(NBLK,),
        in_specs=[
            pl.BlockSpec((2, BLK, WEXT), lambda i: (0, i, 0)),
            pl.BlockSpec((5, HDIM), lambda i: (0, 0)),
            pl.BlockSpec((1, 1, BLK), lambda i: (i, 0, 0)),
            pl.BlockSpec((HDIM, HDIM), lambda i: (0, 0)),
            pl.BlockSpec((G, HDIM), lambda i: (0, 0)),
        ],
        out_specs=pl.BlockSpec((G, HDIM), lambda i: (0, 0)),
        out_shape=jax.ShapeDtypeStruct((G, HDIM), jnp.float32),
        scratch_shapes=[pltpu.VMEM((G, WEXT), jnp.float32)],
    )(acc, p, batch3, wc_pad, bc_pad)


def kernel(x, edge_index, batch_idx, W1, as1, ad1, b1, W2, as2, ad2, b2,
           W3, as3, ad3, b3, g1, be1, rm1, rv1, g2, be2, rm2, rv2,
           g3, be3, rm3, rv3, Wc, bc):
    src3 = edge_index[0].reshape(NTILES, NCHUNK, K)
    dst3 = edge_index[1].reshape(NTILES, NCHUNK, K)
    batch3 = batch_idx.reshape(NBLK, 1, BLK)

    avec1 = jnp.stack([as1, ad1])
    avec2 = jnp.stack([as2, ad2])
    avec3 = jnp.stack([as3, ad3])
    p1 = jnp.stack([b1, g1, be1, rm1, rv1])
    p2 = jnp.stack([b2, g2, be2, rm2, rv2])
    p3 = jnp.stack([b3, g3, be3, rm3, rv3])
    wc_pad = jnp.pad(Wc, ((0, 0), (0, HDIM - C)))
    bc_pad = jnp.broadcast_to(jnp.pad(bc, (0, HDIM - C))[None, :], (G, HDIM))

    def edge(hext, al, alm):
        return _sc_edge(hext, al[:, 0].reshape(NP), al[:, 1].reshape(NP),
                        alm.reshape(8 * HDIM), src3, dst3)

    hext, al, alm = _tc_first(x, W1, avec1)
    acc = edge(hext, al, alm)
    hext, al, alm = _tc_layer(acc, p1, W2, avec2)
    acc = edge(hext, al, alm)
    hext, al, alm = _tc_layer(acc, p2, W3, avec3)
    acc = edge(hext, al, alm)
    out = _tc_pool(acc, p3, batch3, wc_pad, bc_pad)
    return out[:, :C]


# speedup vs baseline: 1.1756x; 1.1756x over previous
import jax
import jax.numpy as jnp
from jax import lax
from jax.experimental import pallas as pl
from jax.experimental.pallas import tpu as pltpu
from jax.experimental.pallas import tpu_sc as plsc

N = 10000
E = 320000
HDIM = 128
NP = 10000
WEXT = 144
NEG = 0.2
BN_EPS = 1e-5
G = 16
C = 10

NCORES = 2
NSUB = 16
NTILES = NCORES * NSUB
EPT = E // NTILES
K = 80
NCHUNK = EPT // K
CPG = 25
NGRP = NCHUNK // CPG
RPS = NP // NSUB
BLK = 2000
NBLK = NP // BLK


def _sc_edge_body(hext_hbm, alphas_hbm, alphad_hbm, alm_hbm, src_hbm,
                  dst_hbm, acc_hbm, alphas_v, alphad_v, alm_v, src_v, dst_v,
                  rows_v, ex_v, acc_sh, gsem):
    c = lax.axis_index("c")
    s = lax.axis_index("s")
    wid = s * NCORES + c

    pltpu.sync_copy(alphas_hbm, alphas_v)
    pltpu.sync_copy(alphad_hbm, alphad_v)
    pltpu.sync_copy(alm_hbm, alm_v)

    def zrow(i, carry):
        for cv in range(WEXT // 16):
            rows_v[i, pl.ds(cv * 16, 16)] = jnp.zeros((16,), jnp.float32)
        return carry

    lax.fori_loop(0, K, zrow, 0)
    nzc = RPS // K
    rem = RPS - nzc * K
    for j in range(nzc):
        pltpu.sync_copy(rows_v, acc_sh.at[pl.ds(s * RPS + j * K, K)])
    pltpu.sync_copy(rows_v.at[pl.ds(0, rem)],
                    acc_sh.at[pl.ds(s * RPS + nzc * K, rem)])
    plsc.subcore_barrier()

    gb = alm_v[pl.ds(0, 16)] + alm_v[pl.ds(512, 16)]
    gbound = jnp.where(gb > 0, gb, NEG * gb)

    def group(gix, carry):
        pltpu.sync_copy(src_hbm.at[wid, pl.ds(gix * CPG, CPG)], src_v)
        pltpu.sync_copy(dst_hbm.at[wid, pl.ds(gix * CPG, CPG)], dst_v)

        def chunk(cix, carry1):
            pltpu.async_copy(hext_hbm.at[src_v.at[cix]], rows_v, gsem).wait()
            for v5 in range(K // 16):
                sidx = src_v[cix, pl.ds(v5 * 16, 16)]
                didx = dst_v[cix, pl.ds(v5 * 16, 16)]
                av = plsc.load_gather(alphas_v, [sidx])
                bv = plsc.load_gather(alphad_v, [didx])
                z = av + bv
                z = jnp.where(z > 0, z, NEG * z)
                ex_v[pl.ds(v5 * 16, 16)] = jnp.exp(z - gbound)
            def srow(r4, carry2):
                for u in range(4):
                    r = r4 * 4 + u
                    bex = plsc.load_gather(ex_v,
                                           [jnp.full((16,), r, jnp.int32)])
                    for cv in range(WEXT // 16):
                        rows_v[r, pl.ds(cv * 16, 16)] = (
                            rows_v[r, pl.ds(cv * 16, 16)] * bex)
                return carry2
            lax.fori_loop(0, K // 4, srow, 0)
            pltpu.sync_copy(rows_v, acc_sh.at[dst_v.at[cix]], add=True)
            return carry1

        lax.fori_loop(0, CPG, chunk, 0)
        return carry

    lax.fori_loop(0, NGRP, group, 0)
    plsc.subcore_barrier()

    pltpu.sync_copy(acc_sh.at[pl.ds(s * RPS, RPS)],
                    acc_hbm.at[c, pl.ds(s * RPS, RPS)])


def _sc_edge(hext, alphas, alphad, alm, src3, dst3):
    mesh = plsc.VectorSubcoreMesh(core_axis_name="c", subcore_axis_name="s",
                                  num_cores=NCORES, num_subcores=NSUB)
    f = pl.kernel(
        _sc_edge_body,
        out_type=jax.ShapeDtypeStruct((NCORES, NP, WEXT), jnp.float32),
        mesh=mesh,
        compiler_params=pltpu.CompilerParams(needs_layout_passes=False,
                                             use_tc_tiling_on_sc=False),
        scratch_types=[
            pltpu.VMEM((NP,), jnp.float32),
            pltpu.VMEM((NP,), jnp.float32),
            pltpu.VMEM((G * 64,), jnp.float32),
            pltpu.VMEM((CPG, K), jnp.int32),
            pltpu.VMEM((CPG, K), jnp.int32),
            pltpu.VMEM((K, WEXT), jnp.float32),
            pltpu.VMEM((128,), jnp.float32),
            pltpu.VMEM_SHARED((NP, WEXT), jnp.float32),
            pltpu.SemaphoreType.DMA,
        ],
    )
    return f(hext, alphas, alphad, alm, src3, dst3)


def _epilogue(a0, a1, p):
    u = a0[:, :HDIM] + a1[:, :HDIM]
    dn = a0[:, HDIM:HDIM + 1] + a1[:, HDIM:HDIM + 1]
    u = u / (dn + 1e-30)
    u = u + p[0:1, :]
    u = (u - p[3:4, :]) * lax.rsqrt(p[4:5, :] + BN_EPS) * p[1:2, :] + p[2:3, :]
    return jnp.maximum(u, 0.0)


def _hext_alpha(i, t, w, avec, hext_ref, al_ref, alm_ref):
    h = jnp.dot(t, w, preferred_element_type=jnp.float32)
    hext_ref[...] = jnp.concatenate(
        [h, jnp.ones((h.shape[0], 1), jnp.float32),
         jnp.zeros((h.shape[0], WEXT - HDIM - 1), jnp.float32)], axis=-1)
    al = lax.dot_general(avec, h, (((1,), (1,)), ((), ())),
                         preferred_element_type=jnp.float32)
    al_ref[...] = al[None]

    @pl.when(i == 0)
    def _():
        alm_ref[...] = jnp.full((8, HDIM), -3e38, jnp.float32)

    bmax = jnp.max(al, axis=1, keepdims=True)
    bmax8 = jnp.concatenate(
        [jnp.broadcast_to(bmax[0:1], (4, HDIM)),
         jnp.broadcast_to(bmax[1:2], (4, HDIM))], axis=0)
    alm_ref[...] = jnp.maximum(alm_ref[...], bmax8)


def _tc_first_body(x_ref, w_ref, avec_ref, hext_ref, al_ref, alm_ref):
    _hext_alpha(pl.program_id(0), x_ref[...], w_ref[...], avec_ref[...],
                hext_ref, al_ref, alm_ref)


def _tc_layer_body(acc_ref, p_ref, w_ref, avec_ref, hext_ref, al_ref,
                   alm_ref):
    t = _epilogue(acc_ref[0], acc_ref[1], p_ref[...])
    _hext_alpha(pl.program_id(0), t, w_ref[...], avec_ref[...],
                hext_ref, al_ref, alm_ref)


def _tc_pool_body(acc_ref, p_ref, batch_ref, wc_ref, bc_ref, out_ref,
                  sums_ref):
    i = pl.program_id(0)

    @pl.when(i == 0)
    def _():
        sums_ref[...] = jnp.zeros_like(sums_ref)

    t = _epilogue(acc_ref[0], acc_ref[1], p_ref[...])
    text = jnp.concatenate(
        [t, jnp.ones((t.shape[0], 1), jnp.float32),
         jnp.zeros((t.shape[0], WEXT - HDIM - 1), jnp.float32)], axis=-1)
    oh = (batch_ref[0] == jnp.arange(G, dtype=jnp.int32)[:, None])
    sums_ref[...] += lax.dot_general(
        oh.astype(jnp.float32), text, (((1,), (0,)), ((), ())),
        preferred_element_type=jnp.float32)

    @pl.when(i == pl.num_programs(0) - 1)
    def _():
        sums = sums_ref[...]
        pooled = sums[:, :HDIM] / jnp.clip(sums[:, HDIM:HDIM + 1], 1.0, None)
        out_ref[...] = (jnp.dot(pooled, wc_ref[...],
                                preferred_element_type=jnp.float32)
                        + bc_ref[...])


_DENSE_OUT = [
    jax.ShapeDtypeStruct((NP, WEXT), jnp.float32),
    jax.ShapeDtypeStruct((NBLK, 2, BLK), jnp.float32),
    jax.ShapeDtypeStruct((8, HDIM), jnp.float32),
]
_DENSE_OUT_SPECS = [
    pl.BlockSpec((BLK, WEXT), lambda i: (i, 0)),
    pl.BlockSpec((1, 2, BLK), lambda i: (i, 0, 0)),
    pl.BlockSpec((8, HDIM), lambda i: (0, 0)),
]


def _tc_first(x, w, avec):
    return pl.pallas_call(
        _tc_first_body,
        grid=(NBLK,),
        in_specs=[
            pl.BlockSpec((BLK, HDIM), lambda i: (i, 0)),
            pl.BlockSpec((HDIM, HDIM), lambda i: (0, 0)),
            pl.BlockSpec((2, HDIM), lambda i: (0, 0)),
        ],
        out_specs=_DENSE_OUT_SPECS,
        out_shape=_DENSE_OUT,
    )(x, w, avec)


def _tc_layer(acc, p, w, avec):
    return pl.pallas_call(
        _tc_layer_body,
        grid=(NBLK,),
        in_specs=[
            pl.BlockSpec((2, BLK, WEXT), lambda i: (0, i, 0)),
            pl.BlockSpec((5, HDIM), lambda i: (0, 0)),
            pl.BlockSpec((HDIM, HDIM), lambda i: (0, 0)),
            pl.BlockSpec((2, HDIM), lambda i: (0, 0)),
        ],
        out_specs=_DENSE_OUT_SPECS,
        out_shape=_DENSE_OUT,
    )(acc, p, w, avec)


def _tc_pool(acc, p, batch3, wc_pad, bc_pad):
    return pl.pallas_call(
        _tc_pool_body,
        grid=(NBLK,),
        in_specs=[
            pl.BlockSpec((2, BLK, WEXT), lambda i: (0, i, 0)),
            pl.BlockSpec((5, HDIM), lambda i: (0, 0)),
            pl.BlockSpec((1, 1, BLK), lambda i: (i, 0, 0)),
            pl.BlockSpec((HDIM, HDIM), lambda i: (0, 0)),
            pl.BlockSpec((G, HDIM), lambda i: (0, 0)),
        ],
        out_specs=pl.BlockSpec((G, HDIM), lambda i: (0, 0)),
        out_shape=jax.ShapeDtypeStruct((G, HDIM), jnp.float32),
        scratch_shapes=[pltpu.VMEM((G, WEXT), jnp.float32)],
    )(acc, p, batch3, wc_pad, bc_pad)


def kernel(x, edge_index, batch_idx, W1, as1, ad1, b1, W2, as2, ad2, b2,
           W3, as3, ad3, b3, g1, be1, rm1, rv1, g2, be2, rm2, rv2,
           g3, be3, rm3, rv3, Wc, bc):
    src3 = edge_index[0].reshape(NTILES, NCHUNK, K)
    dst3 = edge_index[1].reshape(NTILES, NCHUNK, K)
    batch3 = batch_idx.reshape(NBLK, 1, BLK)

    avec1 = jnp.stack([as1, ad1])
    avec2 = jnp.stack([as2, ad2])
    avec3 = jnp.stack([as3, ad3])
    p1 = jnp.stack([b1, g1, be1, rm1, rv1])
    p2 = jnp.stack([b2, g2, be2, rm2, rv2])
    p3 = jnp.stack([b3, g3, be3, rm3, rv3])
    wc_pad = jnp.pad(Wc, ((0, 0), (0, HDIM - C)))
    bc_pad = jnp.broadcast_to(jnp.pad(bc, (0, HDIM - C))[None, :], (G, HDIM))

    def edge(hext, al, alm):
        return _sc_edge(hext, al[:, 0].reshape(NP), al[:, 1].reshape(NP),
                        alm.reshape(8 * HDIM), src3, dst3)

    hext, al, alm = _tc_first(x, W1, avec1)
    acc = edge(hext, al, alm)
    hext, al, alm = _tc_layer(acc, p1, W2, avec2)
    acc = edge(hext, al, alm)
    hext, al, alm = _tc_layer(acc, p2, W3, avec3)
    acc = edge(hext, al, alm)
    out = _tc_pool(acc, p3, batch3, wc_pad, bc_pad)
    return out[:, :C]


# speedup vs baseline: 1.1809x; 1.0045x over previous
import jax
import jax.numpy as jnp
from jax import lax
from jax.experimental import pallas as pl
from jax.experimental.pallas import tpu as pltpu
from jax.experimental.pallas import tpu_sc as plsc

N = 10000
E = 320000
HDIM = 128
NP = 10000
WEXT = 144
NEG = 0.2
BN_EPS = 1e-5
G = 16
C = 10

NCORES = 2
NSUB = 16
NTILES = NCORES * NSUB
EPT = E // NTILES
K = 80
NCHUNK = EPT // K
CPG = 25
NGRP = NCHUNK // CPG
RPS = NP // NSUB
BLK = 5000
NBLK = NP // BLK


def _sc_edge_body(hext_hbm, alphas_hbm, alphad_hbm, alm_hbm, src_hbm,
                  dst_hbm, acc_hbm, alphas_v, alphad_v, alm_v, src_v, dst_v,
                  rows_v, ex_v, acc_sh, gsem):
    c = lax.axis_index("c")
    s = lax.axis_index("s")
    wid = s * NCORES + c

    pltpu.sync_copy(alphas_hbm, alphas_v)
    pltpu.sync_copy(alphad_hbm, alphad_v)
    pltpu.sync_copy(alm_hbm, alm_v)

    def zrow(i, carry):
        for cv in range(WEXT // 16):
            rows_v[i, pl.ds(cv * 16, 16)] = jnp.zeros((16,), jnp.float32)
        return carry

    lax.fori_loop(0, K, zrow, 0)
    nzc = RPS // K
    rem = RPS - nzc * K
    for j in range(nzc):
        pltpu.sync_copy(rows_v, acc_sh.at[pl.ds(s * RPS + j * K, K)])
    pltpu.sync_copy(rows_v.at[pl.ds(0, rem)],
                    acc_sh.at[pl.ds(s * RPS + nzc * K, rem)])
    plsc.subcore_barrier()

    gb = alm_v[pl.ds(0, 16)] + alm_v[pl.ds(512, 16)]
    gbound = jnp.where(gb > 0, gb, NEG * gb)

    def group(gix, carry):
        pltpu.sync_copy(src_hbm.at[wid, pl.ds(gix * CPG, CPG)], src_v)
        pltpu.sync_copy(dst_hbm.at[wid, pl.ds(gix * CPG, CPG)], dst_v)

        def chunk(cix, carry1):
            pltpu.async_copy(hext_hbm.at[src_v.at[cix]], rows_v, gsem).wait()
            for v5 in range(K // 16):
                sidx = src_v[cix, pl.ds(v5 * 16, 16)]
                didx = dst_v[cix, pl.ds(v5 * 16, 16)]
                av = plsc.load_gather(alphas_v, [sidx])
                bv = plsc.load_gather(alphad_v, [didx])
                z = av + bv
                z = jnp.where(z > 0, z, NEG * z)
                ex_v[pl.ds(v5 * 16, 16)] = jnp.exp(z - gbound)
            def srow(r4, carry2):
                for u in range(4):
                    r = r4 * 4 + u
                    bex = plsc.load_gather(ex_v,
                                           [jnp.full((16,), r, jnp.int32)])
                    for cv in range(WEXT // 16):
                        rows_v[r, pl.ds(cv * 16, 16)] = (
                            rows_v[r, pl.ds(cv * 16, 16)] * bex)
                return carry2
            lax.fori_loop(0, K // 4, srow, 0)
            pltpu.sync_copy(rows_v, acc_sh.at[dst_v.at[cix]], add=True)
            return carry1

        lax.fori_loop(0, CPG, chunk, 0)
        return carry

    lax.fori_loop(0, NGRP, group, 0)
    plsc.subcore_barrier()

    pltpu.sync_copy(acc_sh.at[pl.ds(s * RPS, RPS)],
                    acc_hbm.at[c, pl.ds(s * RPS, RPS)])


def _sc_edge(hext, alphas, alphad, alm, src3, dst3):
    mesh = plsc.VectorSubcoreMesh(core_axis_name="c", subcore_axis_name="s",
                                  num_cores=NCORES, num_subcores=NSUB)
    f = pl.kernel(
        _sc_edge_body,
        out_type=jax.ShapeDtypeStruct((NCORES, NP, WEXT), jnp.float32),
        mesh=mesh,
        compiler_params=pltpu.CompilerParams(needs_layout_passes=False,
                                             use_tc_tiling_on_sc=False),
        scratch_types=[
            pltpu.VMEM((NP,), jnp.float32),
            pltpu.VMEM((NP,), jnp.float32),
            pltpu.VMEM((G * 64,), jnp.float32),
            pltpu.VMEM((CPG, K), jnp.int32),
            pltpu.VMEM((CPG, K), jnp.int32),
            pltpu.VMEM((K, WEXT), jnp.float32),
            pltpu.VMEM((128,), jnp.float32),
            pltpu.VMEM_SHARED((NP, WEXT), jnp.float32),
            pltpu.SemaphoreType.DMA,
        ],
    )
    return f(hext, alphas, alphad, alm, src3, dst3)


def _epilogue(a0, a1, p):
    u = a0[:, :HDIM] + a1[:, :HDIM]
    dn = a0[:, HDIM:HDIM + 1] + a1[:, HDIM:HDIM + 1]
    u = u / (dn + 1e-30)
    u = u + p[0:1, :]
    u = (u - p[3:4, :]) * lax.rsqrt(p[4:5, :] + BN_EPS) * p[1:2, :] + p[2:3, :]
    return jnp.maximum(u, 0.0)


def _hext_alpha(i, t, w, avec, hext_ref, al_ref, alm_ref):
    h = jnp.dot(t, w, preferred_element_type=jnp.float32)
    hext_ref[...] = jnp.concatenate(
        [h, jnp.ones((h.shape[0], 1), jnp.float32),
         jnp.zeros((h.shape[0], WEXT - HDIM - 1), jnp.float32)], axis=-1)
    al = lax.dot_general(avec, h, (((1,), (1,)), ((), ())),
                         preferred_element_type=jnp.float32)
    al_ref[...] = al[None]

    @pl.when(i == 0)
    def _():
        alm_ref[...] = jnp.full((8, HDIM), -3e38, jnp.float32)

    bmax = jnp.max(al, axis=1, keepdims=True)
    bmax8 = jnp.concatenate(
        [jnp.broadcast_to(bmax[0:1], (4, HDIM)),
         jnp.broadcast_to(bmax[1:2], (4, HDIM))], axis=0)
    alm_ref[...] = jnp.maximum(alm_ref[...], bmax8)


def _tc_first_body(x_ref, w_ref, avec_ref, hext_ref, al_ref, alm_ref):
    _hext_alpha(pl.program_id(0), x_ref[...], w_ref[...], avec_ref[...],
                hext_ref, al_ref, alm_ref)


def _tc_layer_body(acc_ref, p_ref, w_ref, avec_ref, hext_ref, al_ref,
                   alm_ref):
    t = _epilogue(acc_ref[0], acc_ref[1], p_ref[...])
    _hext_alpha(pl.program_id(0), t, w_ref[...], avec_ref[...],
                hext_ref, al_ref, alm_ref)


def _tc_pool_body(acc_ref, p_ref, batch_ref, wc_ref, bc_ref, out_ref,
                  sums_ref):
    i = pl.program_id(0)

    @pl.when(i == 0)
    def _():
        sums_ref[...] = jnp.zeros_like(sums_ref)

    t = _epilogue(acc_ref[0], acc_ref[1], p_ref[...])
    text = jnp.concatenate(
        [t, jnp.ones((t.shape[0], 1), jnp.float32),
         jnp.zeros((t.shape[0], WEXT - HDIM - 1), jnp.float32)], axis=-1)
    oh = (batch_ref[0] == jnp.arange(G, dtype=jnp.int32)[:, None])
    sums_ref[...] += lax.dot_general(
        oh.astype(jnp.float32), text, (((1,), (0,)), ((), ())),
        preferred_element_type=jnp.float32)

    @pl.when(i == pl.num_programs(0) - 1)
    def _():
        sums = sums_ref[...]
        pooled = sums[:, :HDIM] / jnp.clip(sums[:, HDIM:HDIM + 1], 1.0, None)
        out_ref[...] = (jnp.dot(pooled, wc_ref[...],
                                preferred_element_type=jnp.float32)
                        + bc_ref[...])


_DENSE_OUT = [
    jax.ShapeDtypeStruct((NP, WEXT), jnp.float32),
    jax.ShapeDtypeStruct((NBLK, 2, BLK), jnp.float32),
    jax.ShapeDtypeStruct((8, HDIM), jnp.float32),
]
_DENSE_OUT_SPECS = [
    pl.BlockSpec((BLK, WEXT), lambda i: (i, 0)),
    pl.BlockSpec((1, 2, BLK), lambda i: (i, 0, 0)),
    pl.BlockSpec((8, HDIM), lambda i: (0, 0)),
]


def _tc_first(x, w, avec):
    return pl.pallas_call(
        _tc_first_body,
        grid=(NBLK,),
        in_specs=[
            pl.BlockSpec((BLK, HDIM), lambda i: (i, 0)),
            pl.BlockSpec((HDIM, HDIM), lambda i: (0, 0)),
            pl.BlockSpec((2, HDIM), lambda i: (0, 0)),
        ],
        out_specs=_DENSE_OUT_SPECS,
        out_shape=_DENSE_OUT,
    )(x, w, avec)


def _tc_layer(acc, p, w, avec):
    return pl.pallas_call(
        _tc_layer_body,
        grid=(NBLK,),
        in_specs=[
            pl.BlockSpec((2, BLK, WEXT), lambda i: (0, i, 0)),
            pl.BlockSpec((5, HDIM), lambda i: (0, 0)),
            pl.BlockSpec((HDIM, HDIM), lambda i: (0, 0)),
            pl.BlockSpec((2, HDIM), lambda i: (0, 0)),
        ],
        out_specs=_DENSE_OUT_SPECS,
        out_shape=_DENSE_OUT,
    )(acc, p, w, avec)


def _tc_pool(acc, p, batch3, wc_pad, bc_pad):
    return pl.pallas_call(
        _tc_pool_body,
        grid=(NBLK,),
        in_specs=[
            pl.BlockSpec((2, BLK, WEXT), lambda i: (0, i, 0)),
            pl.BlockSpec((5, HDIM), lambda i: (0, 0)),
            pl.BlockSpec((1, 1, BLK), lambda i: (i, 0, 0)),
            pl.BlockSpec((HDIM, HDIM), lambda i: (0, 0)),
            pl.BlockSpec((G, HDIM), lambda i: (0, 0)),
        ],
        out_specs=pl.BlockSpec((G, HDIM), lambda i: (0, 0)),
        out_shape=jax.ShapeDtypeStruct((G, HDIM), jnp.float32),
        scratch_shapes=[pltpu.VMEM((G, WEXT), jnp.float32)],
    )(acc, p, batch3, wc_pad, bc_pad)


def kernel(x, edge_index, batch_idx, W1, as1, ad1, b1, W2, as2, ad2, b2,
           W3, as3, ad3, b3, g1, be1, rm1, rv1, g2, be2, rm2, rv2,
           g3, be3, rm3, rv3, Wc, bc):
    src3 = edge_index[0].reshape(NTILES, NCHUNK, K)
    dst3 = edge_index[1].reshape(NTILES, NCHUNK, K)
    batch3 = batch_idx.reshape(NBLK, 1, BLK)

    avec1 = jnp.stack([as1, ad1])
    avec2 = jnp.stack([as2, ad2])
    avec3 = jnp.stack([as3, ad3])
    p1 = jnp.stack([b1, g1, be1, rm1, rv1])
    p2 = jnp.stack([b2, g2, be2, rm2, rv2])
    p3 = jnp.stack([b3, g3, be3, rm3, rv3])
    wc_pad = jnp.pad(Wc, ((0, 0), (0, HDIM - C)))
    bc_pad = jnp.broadcast_to(jnp.pad(bc, (0, HDIM - C))[None, :], (G, HDIM))

    def edge(hext, al, alm):
        return _sc_edge(hext, al[:, 0].reshape(NP), al[:, 1].reshape(NP),
                        alm.reshape(8 * HDIM), src3, dst3)

    hext, al, alm = _tc_first(x, W1, avec1)
    acc = edge(hext, al, alm)
    hext, al, alm = _tc_layer(acc, p1, W2, avec2)
    acc = edge(hext, al, alm)
    hext, al, alm = _tc_layer(acc, p2, W3, avec3)
    acc = edge(hext, al, alm)
    out = _tc_pool(acc, p3, batch3, wc_pad, bc_pad)
    return out[:, :C]
